# Initial kernel scaffold; baseline (speedup 1.0000x reference)
import jax
import jax.numpy as jnp
from jax.experimental import pallas as pl


def kernel(job_done, machine_busy_until, truck_location, next_op, proc_times, truck_busy_until, action_mask):
    raise NotImplementedError("write your pallas kernel here")


# speedup vs baseline: 2.5767x; 2.5767x over previous
import functools

import jax
import jax.numpy as jnp
from jax.experimental import pallas as pl

_BB = 64
_IBIG = 1 << 20


def _spt_kernel(nop_ref, pt_ref, tbu_ref, out_ref, *, n_mas, n_trs, n_ops, n_jobs):
    nop = nop_ref[...]
    pt = pt_ref[...]
    tbu = tbu_ref[...]
    bb = nop.shape[0]

    o_iota = jax.lax.broadcasted_iota(jnp.int32, (bb, n_jobs, n_ops), 2)
    j_iota = jax.lax.broadcasted_iota(jnp.int32, (bb, n_jobs, n_ops), 1)
    hit = nop[:, :, None] == o_iota
    firstjob = jnp.min(jnp.where(hit, j_iota, _IBIG), axis=1)

    used = firstjob < _IBIG
    cand = jnp.where((pt == 0.0) | ~used[:, None, :], jnp.inf, pt)
    minval = jnp.min(cand, axis=(1, 2), keepdims=True)
    m_iota = jax.lax.broadcasted_iota(jnp.int32, (bb, n_mas, n_ops), 1)
    key = jnp.where(cand == minval,
                    firstjob[:, None, :] * n_mas + m_iota, _IBIG)
    flat = jnp.min(key, axis=(1, 2))

    tmin = jnp.min(tbu, axis=1, keepdims=True)
    t_iota = jax.lax.broadcasted_iota(jnp.int32, (bb, n_trs), 1)
    tidx = jnp.min(jnp.where(tbu == tmin, t_iota, _IBIG), axis=1)

    action = 1 + flat * n_trs + tidx

    n_cols = out_ref.shape[1]
    col = jax.lax.broadcasted_iota(jnp.int32, (bb, n_cols), 1)
    out_ref[...] = jnp.where(col == action[:, None], 1.0, 0.0).astype(jnp.float32)


def kernel(job_done, machine_busy_until, truck_location, next_op, proc_times,
           truck_busy_until, action_mask):
    B, n_jobs = job_done.shape
    n_mas = machine_busy_until.shape[1]
    n_trs = truck_location.shape[1]
    n_ops = proc_times.shape[2]
    n_cols = 1 + n_jobs * n_mas * n_trs

    body = functools.partial(_spt_kernel, n_mas=n_mas, n_trs=n_trs,
                             n_ops=n_ops, n_jobs=n_jobs)
    logits = pl.pallas_call(
        body,
        grid=(B // _BB,),
        in_specs=[
            pl.BlockSpec((_BB, n_jobs), lambda i: (i, 0)),
            pl.BlockSpec((_BB, n_mas, n_ops), lambda i: (i, 0, 0)),
            pl.BlockSpec((_BB, n_trs), lambda i: (i, 0)),
        ],
        out_specs=pl.BlockSpec((_BB, n_cols), lambda i: (i, 0)),
        out_shape=jax.ShapeDtypeStruct((B, n_cols), jnp.float32),
    )(next_op, proc_times, truck_busy_until)
    return (logits, action_mask)
